# baseline (speedup 1.0000x reference)
import functools

import jax
import jax.numpy as jnp
from jax import lax
from jax.experimental import pallas as pl
from jax.experimental.pallas import tpu as pltpu
from jax.experimental.pallas import tpu_sc as plsc

NC = 2
NS = 16
NW = NC * NS
CHUNK = 64


def _sc_body(x_hbm, idx_hbm, pe_hbm, out_hbm, idx_v, buf, peb, sem):
    wid = lax.axis_index("s") * NC + lax.axis_index("c")
    rows_per_w = x_hbm.shape[0] // NW
    base = wid * rows_per_w
    nchunk = rows_per_w // CHUNK
    d_half = pe_hbm.shape[1]
    n16 = d_half // 16

    def step(g, carry):
        r0 = pl.multiple_of(base + g * CHUNK, CHUNK)
        pltpu.sync_copy(idx_hbm.at[pl.ds(r0, CHUNK)], idx_v)
        pltpu.sync_copy(x_hbm.at[pl.ds(r0, CHUNK)], buf)
        pltpu.async_copy(pe_hbm.at[idx_v], peb, sem).wait()

        def row(r, c2):
            for c in range(n16):
                sl = pl.ds(c * 16, 16)
                plsc.addupdate(buf.at[r, sl], peb[r, sl])
            return c2

        lax.fori_loop(0, CHUNK, row, 0)
        pltpu.sync_copy(buf, out_hbm.at[pl.ds(r0, CHUNK)])
        return carry

    lax.fori_loop(0, nchunk, step, 0)


@functools.cache
def _build(rows, d_half):
    mesh = plsc.VectorSubcoreMesh(core_axis_name="c", subcore_axis_name="s")
    return pl.kernel(
        _sc_body,
        out_type=jax.ShapeDtypeStruct((rows, d_half), jnp.float32),
        mesh=mesh,
        scratch_types=[
            pltpu.VMEM((CHUNK,), jnp.int32),
            pltpu.VMEM((CHUNK, d_half), jnp.float32),
            pltpu.VMEM((CHUNK, d_half), jnp.float32),
            pltpu.SemaphoreType.DMA,
        ],
    )


@jax.jit
def kernel(x, parents, pe):
    Bx, Sx, D = x.shape
    d_half = pe.shape[1]
    abs_idx = lax.broadcasted_iota(jnp.int32, (Bx, Sx), 1)
    idx_all = jnp.stack([abs_idx, parents.astype(jnp.int32)], axis=-1).reshape(-1)
    x_flat = x.reshape(Bx * Sx * 2, d_half)
    out = _build(Bx * Sx * 2, d_half)(x_flat, idx_all, pe)
    return out.reshape(Bx, Sx, D)


# speedup vs baseline: 1.0236x; 1.0236x over previous
import functools

import jax
import jax.numpy as jnp
from jax import lax
from jax.experimental import pallas as pl
from jax.experimental.pallas import tpu as pltpu
from jax.experimental.pallas import tpu_sc as plsc

NC = 2
NS = 16
NW = NC * NS
CHUNK = 32


def _sc_body(x_hbm, idx_hbm, pe_hbm, out_hbm,
             idx_v, buf0, buf1, peb0, peb1, sx0, sx1, sp0, sp1, so0, so1):
    wid = lax.axis_index("s") * NC + lax.axis_index("c")
    rows_per_w = x_hbm.shape[0] // NW
    base = pl.multiple_of(wid * rows_per_w, rows_per_w)
    nchunk = rows_per_w // CHUNK
    d_half = pe_hbm.shape[1]
    n16 = d_half // 16

    bufs = [buf0, buf1]
    pebs = [peb0, peb1]
    sx = [sx0, sx1]
    sp = [sp0, sp1]
    so = [so0, so1]

    pltpu.sync_copy(idx_hbm.at[pl.ds(base, rows_per_w)], idx_v)

    def issue(g):
        b = g & 1
        r0 = pl.multiple_of(base + g * CHUNK, CHUNK)
        dx = pltpu.async_copy(x_hbm.at[pl.ds(r0, CHUNK)], bufs[b], sx[b])
        dp = pltpu.async_copy(pe_hbm.at[idx_v.at[pl.ds(g * CHUNK, CHUNK)]],
                              pebs[b], sp[b])
        return dx, dp

    out_d = [None, None]
    cur = issue(0)
    for g in range(nchunk):
        b = g & 1
        nxt = None
        if g + 1 < nchunk:
            nb = (g + 1) & 1
            if out_d[nb] is not None:
                out_d[nb].wait()
                out_d[nb] = None
            nxt = issue(g + 1)
        cur[0].wait()
        cur[1].wait()
        buf = bufs[b]
        peb = pebs[b]

        def row(r, carry, buf=buf, peb=peb):
            for c in range(n16):
                sl = pl.ds(c * 16, 16)
                plsc.addupdate(buf.at[r, sl], peb[r, sl])
            return carry

        lax.fori_loop(0, CHUNK, row, 0)
        r0 = pl.multiple_of(base + g * CHUNK, CHUNK)
        out_d[b] = pltpu.async_copy(buf, out_hbm.at[pl.ds(r0, CHUNK)], so[b])
        cur = nxt
    for d in out_d:
        if d is not None:
            d.wait()


@functools.cache
def _build(rows, d_half):
    mesh = plsc.VectorSubcoreMesh(core_axis_name="c", subcore_axis_name="s")
    rows_per_w = rows // NW
    return pl.kernel(
        _sc_body,
        out_type=jax.ShapeDtypeStruct((rows, d_half), jnp.float32),
        mesh=mesh,
        scratch_types=[
            pltpu.VMEM((rows_per_w,), jnp.int32),
            pltpu.VMEM((CHUNK, d_half), jnp.float32),
            pltpu.VMEM((CHUNK, d_half), jnp.float32),
            pltpu.VMEM((CHUNK, d_half), jnp.float32),
            pltpu.VMEM((CHUNK, d_half), jnp.float32),
            pltpu.SemaphoreType.DMA,
            pltpu.SemaphoreType.DMA,
            pltpu.SemaphoreType.DMA,
            pltpu.SemaphoreType.DMA,
            pltpu.SemaphoreType.DMA,
            pltpu.SemaphoreType.DMA,
        ],
    )


@jax.jit
def kernel(x, parents, pe):
    Bx, Sx, D = x.shape
    d_half = pe.shape[1]
    abs_idx = lax.broadcasted_iota(jnp.int32, (Bx, Sx), 1)
    idx_all = jnp.stack([abs_idx, parents.astype(jnp.int32)], axis=-1).reshape(-1)
    x_flat = x.reshape(Bx * Sx * 2, d_half)
    out = _build(Bx * Sx * 2, d_half)(x_flat, idx_all, pe)
    return out.reshape(Bx, Sx, D)


# speedup vs baseline: 1.6340x; 1.5963x over previous
import functools

import jax
import jax.numpy as jnp
from jax import lax
from jax.experimental import pallas as pl
from jax.experimental.pallas import tpu as pltpu
from jax.experimental.pallas import tpu_sc as plsc

NC = 2
NS = 16
NW = NC * NS
CHUNK = 16


def _sc_body(s_len, x_hbm, par_hbm, pe_hbm, out_hbm,
             idx_v, buf0, buf1, pa0, pa1, pb0, pb1,
             sx0, sx1, sa0, sa1, sp0, sp1, so0, so1):
    wid = lax.axis_index("s") * NC + lax.axis_index("c")
    rows_per_w = x_hbm.shape[0] // NW
    base = pl.multiple_of(wid * rows_per_w, rows_per_w)
    nchunk = rows_per_w // CHUNK
    d_half = pe_hbm.shape[1]
    n16 = d_half // 16

    bufs = [buf0, buf1]
    pas = [pa0, pa1]
    pbs = [pb0, pb1]
    sx = [sx0, sx1]
    sa = [sa0, sa1]
    sp = [sp0, sp1]
    so = [so0, so1]

    pos_base = lax.rem(base, s_len)

    pltpu.sync_copy(par_hbm.at[pl.ds(base, rows_per_w)], idx_v)

    def issue(g):
        b = g & 1
        r0 = pl.multiple_of(base + g * CHUNK, CHUNK)
        p0 = pl.multiple_of(pos_base + g * CHUNK, CHUNK)
        dx = pltpu.async_copy(x_hbm.at[pl.ds(r0, CHUNK)], bufs[b], sx[b])
        da = pltpu.async_copy(pe_hbm.at[pl.ds(p0, CHUNK)], pas[b], sa[b])
        dp = pltpu.async_copy(pe_hbm.at[idx_v.at[pl.ds(g * CHUNK, CHUNK)]],
                              pbs[b], sp[b])
        return dx, da, dp

    out_d = [None, None]
    cur = issue(0)
    for g in range(nchunk):
        b = g & 1
        nxt = None
        if g + 1 < nchunk:
            nb = (g + 1) & 1
            if out_d[nb] is not None:
                out_d[nb].wait()
                out_d[nb] = None
            nxt = issue(g + 1)
        for d in cur:
            d.wait()
        buf = bufs[b]
        pa = pas[b]
        pb = pbs[b]

        def row(r, carry, buf=buf, pa=pa, pb=pb):
            def cgrp(cg, c2):
                o = cg * 128
                for k in range(8):
                    sl = pl.ds(o + k * 16, 16)
                    plsc.addupdate(buf.at[r, sl], pa[r, sl])
                    plsc.addupdate(buf.at[r, pl.ds(d_half + o + k * 16, 16)],
                                   pb[r, sl])
                return c2
            lax.fori_loop(0, n16 // 8, cgrp, 0)
            return carry

        lax.fori_loop(0, CHUNK, row, 0)
        r0 = pl.multiple_of(base + g * CHUNK, CHUNK)
        out_d[b] = pltpu.async_copy(buf, out_hbm.at[pl.ds(r0, CHUNK)], so[b])
        cur = nxt
    for d in out_d:
        if d is not None:
            d.wait()


@functools.cache
def _build(rows, s_len, d_model, d_half):
    mesh = plsc.VectorSubcoreMesh(core_axis_name="c", subcore_axis_name="s")
    rows_per_w = rows // NW
    return pl.kernel(
        functools.partial(_sc_body, s_len),
        out_type=jax.ShapeDtypeStruct((rows, d_model), jnp.float32),
        mesh=mesh,
        scratch_types=[
            pltpu.VMEM((rows_per_w,), jnp.int32),
            pltpu.VMEM((CHUNK, d_model), jnp.float32),
            pltpu.VMEM((CHUNK, d_model), jnp.float32),
            pltpu.VMEM((CHUNK, d_half), jnp.float32),
            pltpu.VMEM((CHUNK, d_half), jnp.float32),
            pltpu.VMEM((CHUNK, d_half), jnp.float32),
            pltpu.VMEM((CHUNK, d_half), jnp.float32),
        ] + [pltpu.SemaphoreType.DMA] * 8,
    )


@jax.jit
def kernel(x, parents, pe):
    Bx, Sx, D = x.shape
    d_half = pe.shape[1]
    x_flat = x.reshape(Bx * Sx, D)
    par_flat = parents.astype(jnp.int32).reshape(-1)
    out = _build(Bx * Sx, Sx, D, d_half)(x_flat, par_flat, pe)
    return out.reshape(Bx, Sx, D)


# speedup vs baseline: 2.7610x; 1.6897x over previous
import functools

import jax
import jax.numpy as jnp
from jax import lax
from jax.experimental import pallas as pl
from jax.experimental.pallas import tpu as pltpu
from jax.experimental.pallas import tpu_sc as plsc

NC = 2
NS = 16
NW = NC * NS
CHUNK = 64
ROWBLK = 512


def _sc_gather_body(par_hbm, pe_hbm, out_hbm, idx_v, pb0, pb1,
                    sp0, sp1, so0, so1):
    wid = lax.axis_index("s") * NC + lax.axis_index("c")
    rows_per_w = par_hbm.shape[0] // NW
    base = pl.multiple_of(wid * rows_per_w, rows_per_w)
    nchunk = rows_per_w // CHUNK

    pbs = [pb0, pb1]
    sp = [sp0, sp1]
    so = [so0, so1]

    pltpu.sync_copy(par_hbm.at[pl.ds(base, rows_per_w)], idx_v)

    def issue(g):
        b = g & 1
        return pltpu.async_copy(pe_hbm.at[idx_v.at[pl.ds(g * CHUNK, CHUNK)]],
                                pbs[b], sp[b])

    out_d = [None, None]
    cur = issue(0)
    for g in range(nchunk):
        b = g & 1
        nxt = None
        if g + 1 < nchunk:
            nb = (g + 1) & 1
            if out_d[nb] is not None:
                out_d[nb].wait()
                out_d[nb] = None
            nxt = issue(g + 1)
        cur.wait()
        r0 = pl.multiple_of(base + g * CHUNK, CHUNK)
        out_d[b] = pltpu.async_copy(pbs[b], out_hbm.at[pl.ds(r0, CHUNK)], so[b])
        cur = nxt
    for d in out_d:
        if d is not None:
            d.wait()


@functools.cache
def _build_gather(rows, d_half):
    mesh = plsc.VectorSubcoreMesh(core_axis_name="c", subcore_axis_name="s")
    rows_per_w = rows // NW
    return pl.kernel(
        _sc_gather_body,
        out_type=jax.ShapeDtypeStruct((rows, d_half), jnp.float32),
        mesh=mesh,
        scratch_types=[
            pltpu.VMEM((rows_per_w,), jnp.int32),
            pltpu.VMEM((CHUNK, d_half), jnp.float32),
            pltpu.VMEM((CHUNK, d_half), jnp.float32),
        ] + [pltpu.SemaphoreType.DMA] * 4,
    )


def _tc_add_body(x_ref, pe_ref, g_ref, out_ref):
    d_half = pe_ref.shape[1]
    out_ref[:, :d_half] = x_ref[:, :d_half] + pe_ref[...]
    out_ref[:, d_half:] = x_ref[:, d_half:] + g_ref[...]


@functools.cache
def _build_add(rows, s_len, d_model, d_half):
    nblk = rows // ROWBLK
    s_blk = s_len // ROWBLK
    return pl.pallas_call(
        _tc_add_body,
        grid=(nblk,),
        in_specs=[
            pl.BlockSpec((ROWBLK, d_model), lambda i: (i, 0)),
            pl.BlockSpec((ROWBLK, d_half), lambda i: (lax.rem(i, s_blk), 0)),
            pl.BlockSpec((ROWBLK, d_half), lambda i: (i, 0)),
        ],
        out_specs=pl.BlockSpec((ROWBLK, d_model), lambda i: (i, 0)),
        out_shape=jax.ShapeDtypeStruct((rows, d_model), jnp.float32),
        compiler_params=pltpu.CompilerParams(
            dimension_semantics=("arbitrary",),
        ),
    )


@jax.jit
def kernel(x, parents, pe):
    Bx, Sx, D = x.shape
    d_half = pe.shape[1]
    rows = Bx * Sx
    x_flat = x.reshape(rows, D)
    par_flat = parents.astype(jnp.int32).reshape(-1)
    gathered = _build_gather(rows, d_half)(par_flat, pe)
    out = _build_add(rows, Sx, D, d_half)(x_flat, pe, gathered)
    return out.reshape(Bx, Sx, D)


# speedup vs baseline: 2.8804x; 1.0432x over previous
import functools

import jax
import jax.numpy as jnp
from jax import lax
from jax.experimental import pallas as pl
from jax.experimental.pallas import tpu as pltpu
from jax.experimental.pallas import tpu_sc as plsc

NC = 2
NS = 16
NW = NC * NS
CHUNK = 64
ROWBLK = 512


def _sc_gather_body(par_hbm, pe_hbm, out_hbm, idx_v, pb0, pb1,
                    sp0, sp1, so0, so1):
    wid = lax.axis_index("s") * NC + lax.axis_index("c")
    rows_per_w = par_hbm.shape[0] // NW
    base = pl.multiple_of(wid * rows_per_w, rows_per_w)
    nchunk = rows_per_w // CHUNK

    pbs = [pb0, pb1]
    sp = [sp0, sp1]
    so = [so0, so1]

    pltpu.sync_copy(par_hbm.at[pl.ds(base, rows_per_w)], idx_v)

    def issue(g):
        b = g & 1
        return pltpu.async_copy(pe_hbm.at[idx_v.at[pl.ds(g * CHUNK, CHUNK)]],
                                pbs[b], sp[b])

    out_d = [None, None]
    cur = issue(0)
    for g in range(nchunk):
        b = g & 1
        nxt = None
        if g + 1 < nchunk:
            nb = (g + 1) & 1
            if out_d[nb] is not None:
                out_d[nb].wait()
                out_d[nb] = None
            nxt = issue(g + 1)
        cur.wait()
        r0 = pl.multiple_of(base + g * CHUNK, CHUNK)
        out_d[b] = pltpu.async_copy(pbs[b], out_hbm.at[pl.ds(r0, CHUNK)], so[b])
        cur = nxt
    for d in out_d:
        if d is not None:
            d.wait()


@functools.cache
def _build_gather(rows, d_half):
    mesh = plsc.VectorSubcoreMesh(core_axis_name="c", subcore_axis_name="s")
    rows_per_w = rows // NW
    return pl.kernel(
        _sc_gather_body,
        out_type=jax.ShapeDtypeStruct((rows, d_half), jnp.float32),
        mesh=mesh,
        scratch_types=[
            pltpu.VMEM((rows_per_w,), jnp.int32),
            pltpu.VMEM((CHUNK, d_half), jnp.float32),
            pltpu.VMEM((CHUNK, d_half), jnp.float32),
        ] + [pltpu.SemaphoreType.DMA] * 4,
    )


def _tc_add_body(x_ref, pe_ref, g_ref, out_ref):
    d_half = pe_ref.shape[1]
    out_ref[:, :d_half] = x_ref[:, :d_half] + pe_ref[...]
    out_ref[:, d_half:] = x_ref[:, d_half:] + g_ref[...]


@functools.cache
def _build_add(rows, s_len, d_model, d_half):
    nbatch = rows // s_len
    s_blk = s_len // ROWBLK
    return pl.pallas_call(
        _tc_add_body,
        grid=(s_blk, nbatch),
        in_specs=[
            pl.BlockSpec((ROWBLK, d_model), lambda j, b: (b * s_blk + j, 0)),
            pl.BlockSpec((ROWBLK, d_half), lambda j, b: (j, 0)),
            pl.BlockSpec((ROWBLK, d_half), lambda j, b: (b * s_blk + j, 0)),
        ],
        out_specs=pl.BlockSpec((ROWBLK, d_model), lambda j, b: (b * s_blk + j, 0)),
        out_shape=jax.ShapeDtypeStruct((rows, d_model), jnp.float32),
        compiler_params=pltpu.CompilerParams(
            dimension_semantics=("arbitrary", "arbitrary"),
        ),
    )


@jax.jit
def kernel(x, parents, pe):
    Bx, Sx, D = x.shape
    d_half = pe.shape[1]
    rows = Bx * Sx
    x_flat = x.reshape(rows, D)
    par_flat = parents.astype(jnp.int32).reshape(-1)
    gathered = _build_gather(rows, d_half)(par_flat, pe)
    out = _build_add(rows, Sx, D, d_half)(x_flat, pe, gathered)
    return out.reshape(Bx, Sx, D)


# speedup vs baseline: 3.0094x; 1.0448x over previous
import functools

import jax
import jax.numpy as jnp
from jax import lax
from jax.experimental import pallas as pl
from jax.experimental.pallas import tpu as pltpu
from jax.experimental.pallas import tpu_sc as plsc

NC = 2
NS = 16
NW = NC * NS
CHUNK = 64
ROWBLK = 1024


def _sc_gather_body(par_hbm, pe_hbm, out_hbm, idx_v, pb0, pb1,
                    sp0, sp1, so0, so1):
    wid = lax.axis_index("s") * NC + lax.axis_index("c")
    rows_per_w = par_hbm.shape[0] // NW
    base = pl.multiple_of(wid * rows_per_w, rows_per_w)
    nchunk = rows_per_w // CHUNK

    pbs = [pb0, pb1]
    sp = [sp0, sp1]
    so = [so0, so1]

    pltpu.sync_copy(par_hbm.at[pl.ds(base, rows_per_w)], idx_v)

    def issue(g):
        b = g & 1
        return pltpu.async_copy(pe_hbm.at[idx_v.at[pl.ds(g * CHUNK, CHUNK)]],
                                pbs[b], sp[b])

    out_d = [None, None]
    cur = issue(0)
    for g in range(nchunk):
        b = g & 1
        nxt = None
        if g + 1 < nchunk:
            nb = (g + 1) & 1
            if out_d[nb] is not None:
                out_d[nb].wait()
                out_d[nb] = None
            nxt = issue(g + 1)
        cur.wait()
        r0 = pl.multiple_of(base + g * CHUNK, CHUNK)
        out_d[b] = pltpu.async_copy(pbs[b], out_hbm.at[pl.ds(r0, CHUNK)], so[b])
        cur = nxt
    for d in out_d:
        if d is not None:
            d.wait()


@functools.cache
def _build_gather(rows, d_half):
    mesh = plsc.VectorSubcoreMesh(core_axis_name="c", subcore_axis_name="s")
    rows_per_w = rows // NW
    return pl.kernel(
        _sc_gather_body,
        out_type=jax.ShapeDtypeStruct((rows, d_half), jnp.float32),
        mesh=mesh,
        scratch_types=[
            pltpu.VMEM((rows_per_w,), jnp.int32),
            pltpu.VMEM((CHUNK, d_half), jnp.float32),
            pltpu.VMEM((CHUNK, d_half), jnp.float32),
        ] + [pltpu.SemaphoreType.DMA] * 4,
    )


def _tc_add_body(x_ref, pe_ref, g_ref, out_ref):
    d_half = pe_ref.shape[1]
    out_ref[:, :d_half] = x_ref[:, :d_half] + pe_ref[...]
    out_ref[:, d_half:] = x_ref[:, d_half:] + g_ref[...]


@functools.cache
def _build_add(rows, s_len, d_model, d_half):
    nbatch = rows // s_len
    s_blk = s_len // ROWBLK
    return pl.pallas_call(
        _tc_add_body,
        grid=(s_blk, nbatch),
        in_specs=[
            pl.BlockSpec((ROWBLK, d_model), lambda j, b: (b * s_blk + j, 0)),
            pl.BlockSpec((ROWBLK, d_half), lambda j, b: (j, 0)),
            pl.BlockSpec((ROWBLK, d_half), lambda j, b: (b * s_blk + j, 0)),
        ],
        out_specs=pl.BlockSpec((ROWBLK, d_model), lambda j, b: (b * s_blk + j, 0)),
        out_shape=jax.ShapeDtypeStruct((rows, d_model), jnp.float32),
        compiler_params=pltpu.CompilerParams(
            dimension_semantics=("arbitrary", "arbitrary"),
        ),
    )


@jax.jit
def kernel(x, parents, pe):
    Bx, Sx, D = x.shape
    d_half = pe.shape[1]
    rows = Bx * Sx
    x_flat = x.reshape(rows, D)
    par_flat = parents.astype(jnp.int32).reshape(-1)
    gathered = _build_gather(rows, d_half)(par_flat, pe)
    out = _build_add(rows, Sx, D, d_half)(x_flat, pe, gathered)
    return out.reshape(Bx, Sx, D)
